# Initial kernel scaffold; baseline (speedup 1.0000x reference)
import jax
import jax.numpy as jnp
from jax.experimental import pallas as pl


def kernel(x, enc_w1, enc_b1, enc_g1, enc_beta1, enc_w2, enc_b2, enc_g2, enc_beta2, ra_w, ra_b, wa_w, wa_b, wv_w, wv_b, mm_w, mm_b, mn_g, mn_beta, wr_w, wr_b, wz_w, wz_b, wn_w, wn_b, hn_g, hn_beta, dec_w1, dec_b1, dec_g, dec_beta, dec_w2, dec_b2):
    raise NotImplementedError("write your pallas kernel here")


# speedup vs baseline: 2.7213x; 2.7213x over previous
import functools

import jax
import jax.numpy as jnp
from jax.experimental import pallas as pl
from jax.experimental.pallas import tpu as pltpu

_SQRT2 = 1.4142135623730951

_dot = functools.partial(jnp.dot, preferred_element_type=jnp.float32)


def _gelu(x):
    return 0.5 * x * (1.0 + jax.lax.erf(x / _SQRT2))


def _ln_rows(x, g, b):
    m = x.mean(axis=-1, keepdims=True)
    v = jnp.mean((x - m) ** 2, axis=-1, keepdims=True)
    return (x - m) * jax.lax.rsqrt(v + 1e-5) * g + b


def _ln0(xT, g, b):
    m = xT.mean(axis=0, keepdims=True)
    v = jnp.mean((xT - m) ** 2, axis=0, keepdims=True)
    return (xT - m) * jax.lax.rsqrt(v + 1e-5) * g + b


def _sm0(x):
    m = jnp.max(x, axis=0, keepdims=True)
    e = jnp.exp(x - m)
    return e / jnp.sum(e, axis=0, keepdims=True)


def _colmv(w, c):
    return jnp.sum(w * c.T, axis=-1, keepdims=True)


def _enc_kernel(x_ref, w1t_ref, b1_ref, g1_ref, be1_ref,
                w2t_ref, b2_ref, g2_ref, be2_ref,
                waq_ref, wab_ref, wvq_ref, wvb_ref,
                zt_ref, aacc_ref, vacc_ref):
    j = pl.program_id(1)
    h1 = _dot(x_ref[...], w1t_ref[...]) + b1_ref[...]
    h1 = _ln_rows(_gelu(h1), g1_ref[...], be1_ref[...])
    z = _ln_rows(_dot(h1, w2t_ref[...]) + b2_ref[...], g2_ref[...], be2_ref[...])
    zt = z.T
    zt_ref[...] = zt
    wa = _sm0(_dot(waq_ref[...], zt) + wab_ref[...])
    wv = jnp.tanh(_dot(wvq_ref[...], zt) + wvb_ref[...])

    @pl.when(j == 0)
    def _():
        aacc_ref[...] = jnp.zeros_like(aacc_ref)
        vacc_ref[...] = jnp.zeros_like(vacc_ref)

    aacc_ref[0] += jnp.sum(wa, axis=1, keepdims=True)
    vacc_ref[0] += jnp.sum(wv, axis=1, keepdims=True)


def _step1_kernel(z_ref, cellsT_ref, rab_ref,
                  mmr_ref, mmb_ref, mng_ref, mnb_ref,
                  wrzz_ref, wrzm_ref, brz_ref,
                  wnz_ref, wnm_ref, bn_ref,
                  hng_ref, hnb_ref,
                  waq_ref, wab_ref, wvq_ref, wvb_ref,
                  h_ref, aacc_ref, vacc_ref):
    j = pl.program_id(1)
    z = z_ref[...]
    ra = _sm0(rab_ref[...])
    rd = _colmv(cellsT_ref[...], ra)
    mem = _ln0(jnp.tanh(_colmv(mmr_ref[...], rd) + mmb_ref[...]),
               mng_ref[...], mnb_ref[...])
    cm = _colmv(wrzm_ref[...], mem) + brz_ref[...]
    pre = _dot(wrzz_ref[...], z) + cm
    gz = jax.nn.sigmoid(pre[64:])
    n = jnp.tanh(_dot(wnz_ref[...], z) + (_colmv(wnm_ref[...], mem) + bn_ref[...]))
    h = _ln0(gz * n, hng_ref[...], hnb_ref[...])
    h_ref[...] = h
    wa = _sm0(_dot(waq_ref[...], h) + wab_ref[...])
    wv = jnp.tanh(_dot(wvq_ref[...], h) + wvb_ref[...])

    @pl.when(j == 0)
    def _():
        aacc_ref[...] = jnp.zeros_like(aacc_ref)
        vacc_ref[...] = jnp.zeros_like(vacc_ref)

    aacc_ref[0] += jnp.sum(wa, axis=1, keepdims=True)
    vacc_ref[0] += jnp.sum(wv, axis=1, keepdims=True)


def _step_core(z, h, cellsT_ref, raq_ref, rab_ref,
               mmh_ref, mmr_ref, mmb_ref, mng_ref, mnb_ref,
               wrzz_ref, wrzm_ref, wrzh_ref, brz_ref,
               wnz_ref, wnm_ref, wnh_ref, bn_ref,
               hng_ref, hnb_ref):
    ra = _sm0(_dot(raq_ref[...], h) + rab_ref[...])
    rd = _dot(cellsT_ref[...], ra)
    mem = _ln0(jnp.tanh(_dot(mmh_ref[...], h) + _dot(mmr_ref[...], rd) + mmb_ref[...]),
               mng_ref[...], mnb_ref[...])
    pre = (_dot(wrzz_ref[...], z) + _dot(wrzm_ref[...], mem)
           + _dot(wrzh_ref[...], h) + brz_ref[...])
    r = jax.nn.sigmoid(pre[:64])
    gz = jax.nn.sigmoid(pre[64:])
    n = jnp.tanh(_dot(wnz_ref[...], z) + _dot(wnm_ref[...], mem)
                 + _dot(wnh_ref[...], r * h) + bn_ref[...])
    return _ln0((1.0 - gz) * h + gz * n, hng_ref[...], hnb_ref[...])


def _step_kernel(z_ref, h_ref, cellsT_ref, raq_ref, rab_ref,
                 mmh_ref, mmr_ref, mmb_ref, mng_ref, mnb_ref,
                 wrzz_ref, wrzm_ref, wrzh_ref, brz_ref,
                 wnz_ref, wnm_ref, wnh_ref, bn_ref,
                 hng_ref, hnb_ref,
                 waq_ref, wab_ref, wvq_ref, wvb_ref,
                 ho_ref, aacc_ref, vacc_ref):
    j = pl.program_id(1)
    h = _step_core(z_ref[...], h_ref[...], cellsT_ref, raq_ref, rab_ref,
                   mmh_ref, mmr_ref, mmb_ref, mng_ref, mnb_ref,
                   wrzz_ref, wrzm_ref, wrzh_ref, brz_ref,
                   wnz_ref, wnm_ref, wnh_ref, bn_ref, hng_ref, hnb_ref)
    ho_ref[...] = h
    wa = _sm0(_dot(waq_ref[...], h) + wab_ref[...])
    wv = jnp.tanh(_dot(wvq_ref[...], h) + wvb_ref[...])

    @pl.when(j == 0)
    def _():
        aacc_ref[...] = jnp.zeros_like(aacc_ref)
        vacc_ref[...] = jnp.zeros_like(vacc_ref)

    aacc_ref[0] += jnp.sum(wa, axis=1, keepdims=True)
    vacc_ref[0] += jnp.sum(wv, axis=1, keepdims=True)


def _final_kernel(z_ref, h_ref, cellsT_ref, raq_ref, rab_ref,
                  mmh_ref, mmr_ref, mmb_ref, mng_ref, mnb_ref,
                  wrzz_ref, wrzm_ref, wrzh_ref, brz_ref,
                  wnz_ref, wnm_ref, wnh_ref, bn_ref,
                  hng_ref, hnb_ref,
                  dw1_ref, db1_ref, dg_ref, dbeta_ref,
                  dw2_ref, db2_ref,
                  out_ref):
    h = _step_core(z_ref[...], h_ref[...], cellsT_ref, raq_ref, rab_ref,
                   mmh_ref, mmr_ref, mmb_ref, mng_ref, mnb_ref,
                   wrzz_ref, wrzm_ref, wrzh_ref, brz_ref,
                   wnz_ref, wnm_ref, wnh_ref, bn_ref, hng_ref, hnb_ref)
    d = _ln0(_gelu(_dot(dw1_ref[...], h) + db1_ref[...]), dg_ref[...], dbeta_ref[...])
    out_ref[...] = _dot(dw2_ref[...], d) + db2_ref[...]


def _full(shape):
    return pl.BlockSpec(shape, lambda c, j: (0, 0))


def _acc_spec(f):
    return pl.BlockSpec((1, f, 1), lambda c, j: (c, 0, 0))


def _batch_spec(f, blk, g):
    return pl.BlockSpec((f, blk), lambda c, j: (0, c * g + j))


def _cells_update(cells, ap, vp, b):
    wa_mean = ap.sum(axis=0)[:, 0] / b
    wv_mean = vp.sum(axis=0)[:, 0] / b
    c2 = cells + wa_mean[:, None] * wv_mean[None, :]
    nrm = jnp.maximum(jnp.linalg.norm(c2, axis=-1, keepdims=True), 1.0)
    return c2 / nrm


_PARAMS = pltpu.CompilerParams(
    dimension_semantics=("parallel", "arbitrary"),
)


def kernel(x, enc_w1, enc_b1, enc_g1, enc_beta1, enc_w2, enc_b2, enc_g2,
           enc_beta2, ra_w, ra_b, wa_w, wa_b, wv_w, wv_b, mm_w, mm_b, mn_g,
           mn_beta, wr_w, wr_b, wz_w, wz_b, wn_w, wn_b, hn_g, hn_beta,
           dec_w1, dec_b1, dec_g, dec_beta, dec_w2, dec_b2,
           interpret=False):
    B, _ = x.shape
    f32 = jnp.float32

    def tern(w):
        return jnp.sign(w) * (jnp.abs(w) > 0.1).astype(w.dtype)

    raq = tern(ra_w)
    waq = tern(wa_w)
    wvq = tern(wv_w)

    col = lambda v: v[:, None].astype(f32)
    row = lambda v: v[None, :].astype(f32)

    w1t, w2t = enc_w1.T, enc_w2.T
    mmh, mmr = mm_w[:, :64], mm_w[:, 64:]
    wrzz = jnp.concatenate([wr_w[:, :64], wz_w[:, :64]], axis=0)
    wrzm = jnp.concatenate([wr_w[:, 64:128], wz_w[:, 64:128]], axis=0)
    wrzh = jnp.concatenate([wr_w[:, 128:], wz_w[:, 128:]], axis=0)
    brz = jnp.concatenate([wr_b, wz_b], axis=0)[:, None]
    wnz, wnm, wnh = wn_w[:, :64], wn_w[:, 64:128], wn_w[:, 128:]

    rab, wab, wvb = col(ra_b), col(wa_b), col(wv_b)
    mmb, mng, mnb = col(mm_b), col(mn_g), col(mn_beta)
    bn, hng, hnb = col(wn_b), col(hn_g), col(hn_beta)
    db1, dg, dbeta, db2 = col(dec_b1), col(dec_g), col(dec_beta), col(dec_b2)

    def sizes(pref):
        blk = pref
        while B % (2 * blk):
            blk //= 2
        return blk, B // (2 * blk)

    eblk, eg = sizes(1024)
    blk, g = sizes(4096)

    zt, a0, v0 = pl.pallas_call(
        _enc_kernel,
        grid=(2, eg),
        in_specs=[
            pl.BlockSpec((eblk, 768), lambda c, j: (c * eg + j, 0)),
            _full((768, 256)), _full((1, 256)), _full((1, 256)), _full((1, 256)),
            _full((256, 64)), _full((1, 64)), _full((1, 64)), _full((1, 64)),
            _full((32, 64)), _full((32, 1)), _full((64, 64)), _full((64, 1)),
        ],
        out_specs=[
            _batch_spec(64, eblk, eg),
            _acc_spec(32), _acc_spec(64),
        ],
        out_shape=[
            jax.ShapeDtypeStruct((64, B), f32),
            jax.ShapeDtypeStruct((2, 32, 1), f32),
            jax.ShapeDtypeStruct((2, 64, 1), f32),
        ],
        compiler_params=_PARAMS,
        name="rc_encoder",
        interpret=interpret,
    )(x, w1t, row(enc_b1), row(enc_g1), row(enc_beta1),
      w2t, row(enc_b2), row(enc_g2), row(enc_beta2),
      waq, wab, wvq, wvb)

    cells = (a0.sum(axis=0)[:, 0] / B)[:, None] * (v0.sum(axis=0)[:, 0] / B)[None, :]

    step_outs = [
        jax.ShapeDtypeStruct((64, B), f32),
        jax.ShapeDtypeStruct((2, 32, 1), f32),
        jax.ShapeDtypeStruct((2, 64, 1), f32),
    ]
    step_out_specs = [_batch_spec(64, blk, g), _acc_spec(32), _acc_spec(64)]
    zb = _batch_spec(64, blk, g)

    h, ap, vp = pl.pallas_call(
        _step1_kernel,
        grid=(2, g),
        in_specs=[
            zb, _full((64, 32)), _full((32, 1)),
            _full((64, 64)), _full((64, 1)), _full((64, 1)), _full((64, 1)),
            _full((128, 64)), _full((128, 64)), _full((128, 1)),
            _full((64, 64)), _full((64, 64)), _full((64, 1)),
            _full((64, 1)), _full((64, 1)),
            _full((32, 64)), _full((32, 1)), _full((64, 64)), _full((64, 1)),
        ],
        out_specs=step_out_specs,
        out_shape=step_outs,
        compiler_params=_PARAMS,
        name="rc_step1",
        interpret=interpret,
    )(zt, cells.T, rab, mmr, mmb, mng, mnb, wrzz, wrzm, brz,
      wnz, wnm, bn, hng, hnb, waq, wab, wvq, wvb)
    cells = _cells_update(cells, ap, vp, B)

    mid = pl.pallas_call(
        _step_kernel,
        grid=(2, g),
        in_specs=[
            zb, zb, _full((64, 32)), _full((32, 64)), _full((32, 1)),
            _full((64, 64)), _full((64, 64)), _full((64, 1)), _full((64, 1)), _full((64, 1)),
            _full((128, 64)), _full((128, 64)), _full((128, 64)), _full((128, 1)),
            _full((64, 64)), _full((64, 64)), _full((64, 64)), _full((64, 1)),
            _full((64, 1)), _full((64, 1)),
            _full((32, 64)), _full((32, 1)), _full((64, 64)), _full((64, 1)),
        ],
        out_specs=step_out_specs,
        out_shape=step_outs,
        compiler_params=_PARAMS,
        name="rc_step",
        interpret=interpret,
    )
    for _ in range(6):
        h, ap, vp = mid(zt, h, cells.T, raq, rab, mmh, mmr, mmb, mng, mnb,
                        wrzz, wrzm, wrzh, brz, wnz, wnm, wnh, bn, hng, hnb,
                        waq, wab, wvq, wvb)
        cells = _cells_update(cells, ap, vp, B)

    outT = pl.pallas_call(
        _final_kernel,
        grid=(2, g),
        in_specs=[
            zb, zb, _full((64, 32)), _full((32, 64)), _full((32, 1)),
            _full((64, 64)), _full((64, 64)), _full((64, 1)), _full((64, 1)), _full((64, 1)),
            _full((128, 64)), _full((128, 64)), _full((128, 64)), _full((128, 1)),
            _full((64, 64)), _full((64, 64)), _full((64, 64)), _full((64, 1)),
            _full((64, 1)), _full((64, 1)),
            _full((64, 64)), _full((64, 1)), _full((64, 1)), _full((64, 1)),
            _full((2, 64)), _full((2, 1)),
        ],
        out_specs=pl.BlockSpec((2, blk), lambda c, j: (0, c * g + j)),
        out_shape=jax.ShapeDtypeStruct((2, B), f32),
        compiler_params=_PARAMS,
        name="rc_final",
        interpret=interpret,
    )(zt, h, cells.T, raq, rab, mmh, mmr, mmb, mng, mnb,
      wrzz, wrzm, wrzh, brz, wnz, wnm, wnh, bn, hng, hnb,
      dec_w1, db1, dg, dbeta, dec_w2, db2)

    return outT.T


# speedup vs baseline: 2.9971x; 1.1014x over previous
import functools

import jax
import jax.numpy as jnp
from jax.experimental import pallas as pl
from jax.experimental.pallas import tpu as pltpu

_SQRT2 = 1.4142135623730951

_dot = functools.partial(jnp.dot, preferred_element_type=jnp.float32)


def _gelu(x):
    return 0.5 * x * (1.0 + jax.lax.erf(x / _SQRT2))


def _ln_rows(x, g, b, ones_col):
    m = _dot(x, ones_col)
    ms = _dot(x * x, ones_col)
    v = ms - m * m
    return (x - m) * jax.lax.rsqrt(v + 1e-5) * g + b


def _ln0(xT, g, b, ones8_row):
    m = _dot(ones8_row, xT)[:1]
    ms = _dot(ones8_row, xT * xT)[:1]
    v = ms - m * m
    return (xT - m) * jax.lax.rsqrt(v + 1e-5) * g + b


def _means(ap_ref, vp_ref, inv_b):
    return ap_ref[...] * inv_b, vp_ref[...] * inv_b


def _cells_next(cprev, am, vm):
    c2 = cprev + vm * am.T
    nrm = jnp.maximum(jnp.sqrt(jnp.sum(c2 * c2, axis=0, keepdims=True)), 1.0)
    return c2 / nrm


def _colmv(w, c):
    return jnp.sum(w * c.T, axis=-1, keepdims=True)


def _enc_kernel(x_ref, w1t_ref, b1_ref, g1_ref, be1_ref,
                w2t_ref, b2_ref, g2_ref, be2_ref,
                waq_ref, wab_ref, wvq_ref, wvb_ref,
                o256_ref, o64_ref, o832_ref,
                zt_ref, aacc_ref, vacc_ref):
    j = pl.program_id(0)
    h1 = _dot(x_ref[...], w1t_ref[...]) + b1_ref[...]
    h1 = _ln_rows(_gelu(h1), g1_ref[...], be1_ref[...], o256_ref[...])
    z = _ln_rows(_dot(h1, w2t_ref[...]) + b2_ref[...],
                 g2_ref[...], be2_ref[...], o64_ref[...])
    zt = z.T
    zt_ref[...] = zt

    ea = jnp.exp(_dot(waq_ref[...], zt) + wab_ref[...])
    sa = _dot(o832_ref[...], ea)[:1]
    wa = ea * (1.0 / sa)
    wv = jnp.tanh(_dot(wvq_ref[...], zt) + wvb_ref[...])

    @pl.when(j == 0)
    def _():
        aacc_ref[...] = jnp.zeros_like(aacc_ref)
        vacc_ref[...] = jnp.zeros_like(vacc_ref)

    aacc_ref[...] += jnp.sum(wa, axis=1, keepdims=True)
    vacc_ref[...] += jnp.sum(wv, axis=1, keepdims=True)


def _tail(h, j, waq_ref, wab_ref, wvq_ref, wvb_ref, o832_ref,
          aacc_ref, vacc_ref):
    ea = jnp.exp(_dot(waq_ref[...], h) + wab_ref[...])
    sa = _dot(o832_ref[...], ea)[:1]
    wa = ea * (1.0 / sa)
    wv = jnp.tanh(_dot(wvq_ref[...], h) + wvb_ref[...])

    @pl.when(j == 0)
    def _():
        aacc_ref[...] = jnp.zeros_like(aacc_ref)
        vacc_ref[...] = jnp.zeros_like(vacc_ref)

    aacc_ref[...] += jnp.sum(wa, axis=1, keepdims=True)
    vacc_ref[...] += jnp.sum(wv, axis=1, keepdims=True)


def _make_step1_kernel(inv_b):
    def _step1_kernel(z_ref, a0_ref, v0_ref, rab_ref,
                      mmr_ref, mmb_ref, mng_ref, mnb_ref,
                      wrzz_ref, wrzm_ref, brz_ref,
                      wnz_ref, wnm_ref, bn_ref,
                      hng_ref, hnb_ref,
                      waq_ref, wab_ref, wvq_ref, wvb_ref,
                      o864_ref, o832_ref,
                      h_ref, aacc_ref, vacc_ref, cm_ref):
        j = pl.program_id(0)
        a0, v0 = _means(a0_ref, v0_ref, inv_b)
        cellsT = v0 * a0.T
        cm_ref[...] = cellsT
        z = z_ref[...]
        e = jnp.exp(rab_ref[...])
        ra = e / jnp.sum(e, axis=0, keepdims=True)
        rd = _colmv(cellsT, ra)
        mem0 = jnp.tanh(_colmv(mmr_ref[...], rd) + mmb_ref[...])
        mu = jnp.mean(mem0, axis=0, keepdims=True)
        var = jnp.mean(mem0 * mem0, axis=0, keepdims=True) - mu * mu
        mem = (mem0 - mu) * jax.lax.rsqrt(var + 1e-5) * mng_ref[...] + mnb_ref[...]
        cmc = _colmv(wrzm_ref[...], mem) + brz_ref[...]
        pre = _dot(wrzz_ref[...], z) + cmc
        gz = jax.nn.sigmoid(pre[64:])
        n = jnp.tanh(_dot(wnz_ref[...], z)
                     + (_colmv(wnm_ref[...], mem) + bn_ref[...]))
        h = _ln0(gz * n, hng_ref[...], hnb_ref[...], o864_ref[...])
        h_ref[...] = h
        _tail(h, j, waq_ref, wab_ref, wvq_ref, wvb_ref, o832_ref,
              aacc_ref, vacc_ref)
    return _step1_kernel


def _step_core(z, h, cellsT, raq_ref, rab_ref,
               mmh_ref, mmr_ref, mmb_ref, mng_ref, mnb_ref,
               wrzz_ref, wrzm_ref, wrzh_ref, brz_ref,
               wnz_ref, wnm_ref, wnh_ref, bn_ref,
               hng_ref, hnb_ref, o864_ref, o832_ref):
    e = jnp.exp(_dot(raq_ref[...], h) + rab_ref[...])
    s = _dot(o832_ref[...], e)[:1]
    rd = _dot(cellsT, e) * (1.0 / s)
    mem = _ln0(jnp.tanh(_dot(mmh_ref[...], h) + _dot(mmr_ref[...], rd)
                        + mmb_ref[...]),
               mng_ref[...], mnb_ref[...], o864_ref)
    pre = (_dot(wrzz_ref[...], z) + _dot(wrzm_ref[...], mem)
           + _dot(wrzh_ref[...], h) + brz_ref[...])
    r = jax.nn.sigmoid(pre[:64])
    gz = jax.nn.sigmoid(pre[64:])
    n = jnp.tanh(_dot(wnz_ref[...], z) + _dot(wnm_ref[...], mem)
                 + _dot(wnh_ref[...], r * h) + bn_ref[...])
    return _ln0((1.0 - gz) * h + gz * n, hng_ref[...], hnb_ref[...], o864_ref)


def _make_step_kernel(inv_b):
    def _step_kernel(z_ref, h_ref, cp_ref, ap_ref, vp_ref, raq_ref, rab_ref,
                     mmh_ref, mmr_ref, mmb_ref, mng_ref, mnb_ref,
                     wrzz_ref, wrzm_ref, wrzh_ref, brz_ref,
                     wnz_ref, wnm_ref, wnh_ref, bn_ref,
                     hng_ref, hnb_ref,
                     waq_ref, wab_ref, wvq_ref, wvb_ref,
                     o864_ref, o832_ref,
                     ho_ref, aacc_ref, vacc_ref, cm_ref):
        j = pl.program_id(0)
        am, vm = _means(ap_ref, vp_ref, inv_b)
        cellsT = _cells_next(cp_ref[...], am, vm)
        cm_ref[...] = cellsT
        h = _step_core(z_ref[...], h_ref[...], cellsT, raq_ref, rab_ref,
                       mmh_ref, mmr_ref, mmb_ref, mng_ref, mnb_ref,
                       wrzz_ref, wrzm_ref, wrzh_ref, brz_ref,
                       wnz_ref, wnm_ref, wnh_ref, bn_ref, hng_ref, hnb_ref,
                       o864_ref[...], o832_ref)
        ho_ref[...] = h
        _tail(h, j, waq_ref, wab_ref, wvq_ref, wvb_ref, o832_ref,
              aacc_ref, vacc_ref)
    return _step_kernel


def _make_final_kernel(inv_b):
    def _final_kernel(z_ref, h_ref, cp_ref, ap_ref, vp_ref, raq_ref, rab_ref,
                      mmh_ref, mmr_ref, mmb_ref, mng_ref, mnb_ref,
                      wrzz_ref, wrzm_ref, wrzh_ref, brz_ref,
                      wnz_ref, wnm_ref, wnh_ref, bn_ref,
                      hng_ref, hnb_ref,
                      dw1_ref, db1_ref, dg_ref, dbeta_ref,
                      dw2_ref, db2_ref,
                      o864_ref, o832_ref,
                      out_ref):
        am, vm = _means(ap_ref, vp_ref, inv_b)
        cellsT = _cells_next(cp_ref[...], am, vm)
        h = _step_core(z_ref[...], h_ref[...], cellsT, raq_ref, rab_ref,
                       mmh_ref, mmr_ref, mmb_ref, mng_ref, mnb_ref,
                       wrzz_ref, wrzm_ref, wrzh_ref, brz_ref,
                       wnz_ref, wnm_ref, wnh_ref, bn_ref, hng_ref, hnb_ref,
                       o864_ref[...], o832_ref)
        d = _ln0(_gelu(_dot(dw1_ref[...], h) + db1_ref[...]),
                 dg_ref[...], dbeta_ref[...], o864_ref[...])
        out_ref[...] = _dot(dw2_ref[...], d) + db2_ref[...]
    return _final_kernel


def _full(shape):
    return pl.BlockSpec(shape, lambda j: tuple(0 for _ in shape))


def _batch_spec(f, blk):
    return pl.BlockSpec((f, blk), lambda j: (0, j))


_PARAMS = pltpu.CompilerParams(
    dimension_semantics=("arbitrary",),
)


def kernel(x, enc_w1, enc_b1, enc_g1, enc_beta1, enc_w2, enc_b2, enc_g2,
           enc_beta2, ra_w, ra_b, wa_w, wa_b, wv_w, wv_b, mm_w, mm_b, mn_g,
           mn_beta, wr_w, wr_b, wz_w, wz_b, wn_w, wn_b, hn_g, hn_beta,
           dec_w1, dec_b1, dec_g, dec_beta, dec_w2, dec_b2,
           interpret=False):
    B, _ = x.shape
    f32 = jnp.float32
    inv_b = 1.0 / B

    def tern(w):
        return jnp.sign(w) * (jnp.abs(w) > 0.1).astype(w.dtype)

    raq = tern(ra_w)
    waq = tern(wa_w)
    wvq = tern(wv_w)

    col = lambda v: v[:, None].astype(f32)
    row = lambda v: v[None, :].astype(f32)

    w1t, w2t = enc_w1.T, enc_w2.T
    mmh, mmr = mm_w[:, :64], mm_w[:, 64:]
    wrzz = jnp.concatenate([wr_w[:, :64], wz_w[:, :64]], axis=0)
    wrzm = jnp.concatenate([wr_w[:, 64:128], wz_w[:, 64:128]], axis=0)
    wrzh = jnp.concatenate([wr_w[:, 128:], wz_w[:, 128:]], axis=0)
    brz = jnp.concatenate([wr_b, wz_b], axis=0)[:, None]
    wnz, wnm, wnh = wn_w[:, :64], wn_w[:, 64:128], wn_w[:, 128:]

    rab, wab, wvb = col(ra_b), col(wa_b), col(wv_b)
    mmb, mng, mnb = col(mm_b), col(mn_g), col(mn_beta)
    bn, hng, hnb = col(wn_b), col(hn_g), col(hn_beta)
    db1, dg, dbeta, db2 = col(dec_b1), col(dec_g), col(dec_beta), col(dec_b2)

    o256 = jnp.full((256, 1), 1.0 / 256, f32)
    o64 = jnp.full((64, 1), 1.0 / 64, f32)
    o864 = jnp.full((8, 64), 1.0 / 64, f32)
    o832 = jnp.ones((8, 32), f32)

    def sizes(pref):
        blk = pref
        while B % blk:
            blk //= 2
        return blk, B // blk

    eblk, eg = sizes(2048)
    blk, g = sizes(8192)

    zt, a0, v0 = pl.pallas_call(
        _enc_kernel,
        grid=(eg,),
        in_specs=[
            pl.BlockSpec((eblk, 768), lambda j: (j, 0)),
            _full((768, 256)), _full((1, 256)), _full((1, 256)), _full((1, 256)),
            _full((256, 64)), _full((1, 64)), _full((1, 64)), _full((1, 64)),
            _full((32, 64)), _full((32, 1)), _full((64, 64)), _full((64, 1)),
            _full((256, 1)), _full((64, 1)), _full((8, 32)),
        ],
        out_specs=[
            _batch_spec(64, eblk),
            _full((32, 1)), _full((64, 1)),
        ],
        out_shape=[
            jax.ShapeDtypeStruct((64, B), f32),
            jax.ShapeDtypeStruct((32, 1), f32),
            jax.ShapeDtypeStruct((64, 1), f32),
        ],
        compiler_params=_PARAMS,
        name="rc_encoder",
        interpret=interpret,
    )(x, w1t, row(enc_b1), row(enc_g1), row(enc_beta1),
      w2t, row(enc_b2), row(enc_g2), row(enc_beta2),
      waq, wab, wvq, wvb, o256, o64, o832)

    step_outs = [
        jax.ShapeDtypeStruct((64, B), f32),
        jax.ShapeDtypeStruct((32, 1), f32),
        jax.ShapeDtypeStruct((64, 1), f32),
        jax.ShapeDtypeStruct((64, 32), f32),
    ]
    step_out_specs = [_batch_spec(64, blk), _full((32, 1)), _full((64, 1)),
                      _full((64, 32))]
    zb = _batch_spec(64, blk)

    h, ap, vp, cm = pl.pallas_call(
        _make_step1_kernel(inv_b),
        grid=(g,),
        in_specs=[
            zb, _full((32, 1)), _full((64, 1)), _full((32, 1)),
            _full((64, 64)), _full((64, 1)), _full((64, 1)), _full((64, 1)),
            _full((128, 64)), _full((128, 64)), _full((128, 1)),
            _full((64, 64)), _full((64, 64)), _full((64, 1)),
            _full((64, 1)), _full((64, 1)),
            _full((32, 64)), _full((32, 1)), _full((64, 64)), _full((64, 1)),
            _full((8, 64)), _full((8, 32)),
        ],
        out_specs=step_out_specs,
        out_shape=step_outs,
        compiler_params=_PARAMS,
        name="rc_step1",
        interpret=interpret,
    )(zt, a0, v0, rab, mmr, mmb, mng, mnb, wrzz, wrzm, brz,
      wnz, wnm, bn, hng, hnb, waq, wab, wvq, wvb, o864, o832)

    mid = pl.pallas_call(
        _make_step_kernel(inv_b),
        grid=(g,),
        in_specs=[
            zb, zb, _full((64, 32)), _full((32, 1)), _full((64, 1)),
            _full((32, 64)), _full((32, 1)),
            _full((64, 64)), _full((64, 64)), _full((64, 1)), _full((64, 1)), _full((64, 1)),
            _full((128, 64)), _full((128, 64)), _full((128, 64)), _full((128, 1)),
            _full((64, 64)), _full((64, 64)), _full((64, 64)), _full((64, 1)),
            _full((64, 1)), _full((64, 1)),
            _full((32, 64)), _full((32, 1)), _full((64, 64)), _full((64, 1)),
            _full((8, 64)), _full((8, 32)),
        ],
        out_specs=step_out_specs,
        out_shape=step_outs,
        compiler_params=_PARAMS,
        name="rc_step",
        interpret=interpret,
    )
    for _ in range(6):
        h, ap, vp, cm = mid(zt, h, cm, ap, vp, raq, rab, mmh, mmr, mmb, mng,
                            mnb, wrzz, wrzm, wrzh, brz, wnz, wnm, wnh, bn,
                            hng, hnb, waq, wab, wvq, wvb, o864, o832)

    outT = pl.pallas_call(
        _make_final_kernel(inv_b),
        grid=(g,),
        in_specs=[
            zb, zb, _full((64, 32)), _full((32, 1)), _full((64, 1)),
            _full((32, 64)), _full((32, 1)),
            _full((64, 64)), _full((64, 64)), _full((64, 1)), _full((64, 1)), _full((64, 1)),
            _full((128, 64)), _full((128, 64)), _full((128, 64)), _full((128, 1)),
            _full((64, 64)), _full((64, 64)), _full((64, 64)), _full((64, 1)),
            _full((64, 1)), _full((64, 1)),
            _full((64, 64)), _full((64, 1)), _full((64, 1)), _full((64, 1)),
            _full((2, 64)), _full((2, 1)),
            _full((8, 64)), _full((8, 32)),
        ],
        out_specs=pl.BlockSpec((2, blk), lambda j: (0, j)),
        out_shape=jax.ShapeDtypeStruct((2, B), f32),
        compiler_params=_PARAMS,
        name="rc_final",
        interpret=interpret,
    )(zt, h, cm, ap, vp, raq, rab, mmh, mmr, mmb, mng, mnb,
      wrzz, wrzm, wrzh, brz, wnz, wnm, wnh, bn, hng, hnb,
      dec_w1, db1, dg, dbeta, dec_w2, db2, o864, o832)

    return outT.T
